# baseline (speedup 1.0000x reference)
import functools

import jax
import jax.numpy as jnp
from jax import lax
from jax.experimental import pallas as pl
from jax.experimental.pallas import tpu as pltpu
from jax.experimental.pallas import tpu_sc as plsc

_NUM_EMB = 1000002
_D = 64
_B = 16384
_NW = 32
_BW = _B // _NW
_LOOK = 3


def _rsqrt_scalar(x):
    i = lax.bitcast_convert_type(x, jnp.int32)
    i = jnp.int32(0x5F3759DF) - lax.shift_right_logical(i, 1)
    y = lax.bitcast_convert_type(i, jnp.float32)
    for _ in range(3):
        y = y * (jnp.float32(1.5) - jnp.float32(0.5) * x * y * y)
    return y


def _sc_body(table_t, nodes_hbm, out_hbm, nv, ring, panel,
             sem0, sem1, sem2, sem3):
    wid = lax.axis_index("s") * 2 + lax.axis_index("c")
    base = wid * _BW
    sems = [sem0, sem1, sem2, sem3]

    pltpu.sync_copy(nodes_hbm.at[pl.ds(base, _BW)], nv.at[pl.ds(0, _BW)])

    def node_at(i):
        return nv[pl.ds(i, 16)][0]

    def fire(i, slot):
        c = lax.shift_right_logical(node_at(i), 7)
        off = pl.multiple_of(c * 128, 128)
        pltpu.make_async_copy(
            table_t.at[:, pl.ds(off, 128)], ring.at[slot], sems[slot]
        ).start()

    def process(i, slot):
        n = node_at(i)
        pltpu.make_async_copy(
            table_t.at[:, pl.ds(0, 128)], ring.at[slot], sems[slot]
        ).wait()
        lane = jnp.broadcast_to(n & 127, (16,))
        acc = jnp.zeros((16,), jnp.float32)
        vals = []
        for k in range(_D // 16):
            row = lax.iota(jnp.int32, 16) + k * 16
            v = plsc.load_gather(ring.at[slot], [row, lane])
            vals.append(v)
            acc = acc + v * v
        r = _rsqrt_scalar(jnp.sum(acc))
        for k in range(_D // 16):
            panel[i, pl.ds(k * 16, 16)] = vals[k] * r

    for j in range(3):
        fire(j, j)

    def main_body(g, _):
        i = g * 4
        for j in range(4):
            fire(i + j + 3, (j + 3) & 3)
            process(i + j, j)
        return 0

    lax.fori_loop(0, _BW // 4 - 1, main_body, 0)
    process(_BW - 4, 0)
    fire(_BW - 1, 3)
    process(_BW - 3, 1)
    process(_BW - 2, 2)
    process(_BW - 1, 3)

    pltpu.sync_copy(panel, out_hbm.at[pl.ds(base, _BW), :])


@jax.jit
def _sc_call(table_t, nodes):
    mesh = plsc.VectorSubcoreMesh(core_axis_name="c", subcore_axis_name="s")
    return pl.kernel(
        _sc_body,
        out_type=jax.ShapeDtypeStruct((_B, 128), jnp.float32),
        mesh=mesh,
        compiler_params=pltpu.CompilerParams(
            needs_layout_passes=False, use_tc_tiling_on_sc=True
        ),
        scratch_types=[
            pltpu.VMEM((_BW + 16,), jnp.int32),
            pltpu.VMEM((_LOOK + 1, _D, 128), jnp.float32),
            pltpu.VMEM((_BW, 128), jnp.float32),
            pltpu.SemaphoreType.DMA,
            pltpu.SemaphoreType.DMA,
            pltpu.SemaphoreType.DMA,
            pltpu.SemaphoreType.DMA,
        ],
    )(table_t, nodes)


def kernel(nodes, table):
    return _sc_call(table.T, nodes)[:, :_D].T


# speedup vs baseline: 1.1567x; 1.1567x over previous
import functools

import jax
import jax.numpy as jnp
from jax import lax
from jax.experimental import pallas as pl
from jax.experimental.pallas import tpu as pltpu
from jax.experimental.pallas import tpu_sc as plsc

_NUM_EMB = 1000002
_D = 64
_B = 16384
_NW = 32
_BW = _B // _NW
_LOOK = 3


def _rsqrt_scalar(x):
    i = lax.bitcast_convert_type(x, jnp.int32)
    i = jnp.int32(0x5F3759DF) - lax.shift_right_logical(i, 1)
    y = lax.bitcast_convert_type(i, jnp.float32)
    for _ in range(3):
        y = y * (jnp.float32(1.5) - jnp.float32(0.5) * x * y * y)
    return y


def _sc_body(table_t, nodes_hbm, out_hbm, nv, ring, panel, gsem):
    wid = lax.axis_index("s") * 2 + lax.axis_index("c")
    base = wid * _BW

    pltpu.sync_copy(nodes_hbm.at[pl.ds(base, _BW)], nv.at[pl.ds(0, _BW)])

    def node_at(i):
        return nv[pl.ds(i, 16)][0]

    def fire(i, slot):
        c = lax.shift_right_logical(node_at(i), 7)
        off = pl.multiple_of(c * 128, 128)
        pltpu.make_async_copy(
            table_t.at[:, pl.ds(off, 128)], ring.at[slot], gsem
        ).start()

    def process(i, slot):
        n = node_at(i)
        pltpu.make_async_copy(
            table_t.at[:, pl.ds(0, 128)], ring.at[0], gsem
        ).wait()
        lane = jnp.broadcast_to(n & 127, (16,))
        acc = jnp.zeros((16,), jnp.float32)
        vals = []
        for k in range(_D // 16):
            row = lax.iota(jnp.int32, 16) + k * 16
            v = plsc.load_gather(ring.at[slot], [row, lane])
            vals.append(v)
            acc = acc + v * v
        r = _rsqrt_scalar(jnp.sum(acc))
        for k in range(_D // 16):
            panel[i, pl.ds(k * 16, 16)] = vals[k] * r

    for i in range(_LOOK):
        fire(i, i)

    def main_body(i, _):
        fire(i + _LOOK, (i + _LOOK) & _LOOK)
        process(i, i & _LOOK)
        return 0

    lax.fori_loop(0, _BW - _LOOK, main_body, 0)

    def tail_body(i, _):
        process(i, i & _LOOK)
        return 0

    lax.fori_loop(_BW - _LOOK, _BW, tail_body, 0)

    pltpu.sync_copy(panel, out_hbm.at[pl.ds(base, _BW), :])


@jax.jit
def _sc_call(table_t, nodes):
    mesh = plsc.VectorSubcoreMesh(core_axis_name="c", subcore_axis_name="s")
    return pl.kernel(
        _sc_body,
        out_type=jax.ShapeDtypeStruct((_B, 128), jnp.float32),
        mesh=mesh,
        compiler_params=pltpu.CompilerParams(
            needs_layout_passes=False, use_tc_tiling_on_sc=True
        ),
        scratch_types=[
            pltpu.VMEM((_BW + 16,), jnp.int32),
            pltpu.VMEM((_LOOK + 1, _D, 128), jnp.float32),
            pltpu.VMEM((_BW, 128), jnp.float32),
            pltpu.SemaphoreType.DMA,
        ],
    )(table_t, nodes)


def kernel(nodes, table):
    return _sc_call(table.T, nodes)[:, :_D].T
